# Initial kernel scaffold; baseline (speedup 1.0000x reference)
import jax
import jax.numpy as jnp
from jax import lax
from jax.experimental import pallas as pl
from jax.experimental.pallas import tpu as pltpu
from jax.experimental.pallas import tpu_sc as plsc

NUM_TOKENS = 16384 * 50
CHUNK = 128
NUM_CHUNKS = NUM_TOKENS // CHUNK
NC, NS = 2, 16
NW = NC * NS
CH_PER_W = NUM_CHUNKS // NW
EMB_D = 32


def _body(idx_hbm, table_hbm, out_hbm, idx_v, rows_v, sem):
    wid = lax.axis_index("s") * NC + lax.axis_index("c")
    ch_base = wid * CH_PER_W
    pltpu.sync_copy(idx_hbm.at[pl.ds(ch_base, CH_PER_W)], idx_v)

    def step(j, _):
        pltpu.async_copy(table_hbm.at[idx_v.at[j]], rows_v, sem).wait()
        pltpu.sync_copy(
            rows_v, out_hbm.at[pl.ds((ch_base + j) * CHUNK, CHUNK)])
        return 0

    lax.fori_loop(0, CH_PER_W, step, 0)


@jax.jit
def _gather(idx2d, table):
    mesh = plsc.VectorSubcoreMesh(core_axis_name="c", subcore_axis_name="s")
    f = pl.kernel(
        _body,
        out_type=jax.ShapeDtypeStruct((NUM_TOKENS, EMB_D), jnp.float32),
        mesh=mesh,
        scratch_types=[
            pltpu.VMEM((CH_PER_W, CHUNK), jnp.int32),
            pltpu.VMEM((CHUNK, EMB_D), jnp.float32),
            pltpu.SemaphoreType.DMA,
        ],
    )
    return f(idx2d, table)


def kernel(token_ids, embedding_matrix):
    idx2d = token_ids.reshape(NUM_CHUNKS, CHUNK).astype(jnp.int32)
    out = _gather(idx2d, embedding_matrix)
    return out.reshape(token_ids.shape + (EMB_D,))


# speedup vs baseline: 1.0241x; 1.0241x over previous
import jax
import jax.numpy as jnp
from jax import lax
from jax.experimental import pallas as pl
from jax.experimental.pallas import tpu as pltpu
from jax.experimental.pallas import tpu_sc as plsc

NUM_TOKENS = 16384 * 50
CHUNK = 128
NUM_CHUNKS = NUM_TOKENS // CHUNK
NC, NS = 2, 16
NW = NC * NS
CH_PER_W = NUM_CHUNKS // NW
EMB_D = 32


def _body(idx_hbm, table_hbm, out_hbm, idx_v, rows_v, sem):
    wid = lax.axis_index("s") * NC + lax.axis_index("c")
    ch_base = wid * CH_PER_W
    pltpu.sync_copy(idx_hbm.at[pl.ds(ch_base, CH_PER_W)], idx_v)

    def step(j, _):
        pltpu.async_copy(table_hbm.at[idx_v.at[j]], rows_v, sem).wait()
        pltpu.sync_copy(
            rows_v, out_hbm.at[pl.ds((ch_base + j) * CHUNK, CHUNK)])
        return 0

    lax.fori_loop(0, CH_PER_W, step, 0)


@jax.jit
def _gather(idx2d, table):
    mesh = plsc.VectorSubcoreMesh(core_axis_name="c", subcore_axis_name="s")
    f = pl.kernel(
        _body,
        out_type=jax.ShapeDtypeStruct((NUM_TOKENS, EMB_D), jnp.float32),
        mesh=mesh,
        scratch_types=[
            pltpu.VMEM((CH_PER_W, CHUNK), jnp.int32),
            pltpu.VMEM((CHUNK, EMB_D), jnp.float32),
            pltpu.SemaphoreType.DMA,
        ],
        compiler_params=pltpu.CompilerParams(use_tc_tiling_on_sc=False),
    )
    return f(idx2d, table)


def kernel(token_ids, embedding_matrix):
    idx2d = token_ids.reshape(NUM_CHUNKS, CHUNK).astype(jnp.int32)
    out = _gather(idx2d, embedding_matrix)
    return out.reshape(token_ids.shape + (EMB_D,))


# speedup vs baseline: 1.1129x; 1.0867x over previous
import jax
import jax.numpy as jnp
from jax import lax
from jax.experimental import pallas as pl
from jax.experimental.pallas import tpu as pltpu
from jax.experimental.pallas import tpu_sc as plsc

NUM_TOKENS = 16384 * 50
CHUNK = 128
NUM_CHUNKS = NUM_TOKENS // CHUNK
NC, NS = 2, 16
NW = NC * NS
CH_PER_W = NUM_CHUNKS // NW
EMB_D = 32
DEPTH = 8
LAG = 4
GROUPS = CH_PER_W // DEPTH


def _body(idx_hbm, table_hbm, out_hbm, idx_v, *rest):
    rows = rest[:DEPTH]
    gsem = rest[DEPTH:2 * DEPTH]
    ssem = rest[2 * DEPTH:3 * DEPTH]
    wid = lax.axis_index("s") * NC + lax.axis_index("c")
    ch_base = wid * CH_PER_W
    tok_base = ch_base * CHUNK
    pltpu.sync_copy(idx_hbm.at[pl.ds(ch_base, CH_PER_W)], idx_v)

    def gather_start(chunk, slot):
        pltpu.async_copy(table_hbm.at[idx_v.at[chunk]], rows[slot],
                         gsem[slot])

    def gather_wait(slot):
        pltpu.make_async_copy(table_hbm.at[pl.ds(0, CHUNK)], rows[slot],
                              gsem[slot]).wait()

    def store_start(chunk, slot):
        pltpu.async_copy(rows[slot],
                         out_hbm.at[pl.ds(tok_base + chunk * CHUNK, CHUNK)],
                         ssem[slot])

    def store_wait(slot):
        pltpu.make_async_copy(rows[slot], out_hbm.at[pl.ds(0, CHUNK)],
                              ssem[slot]).wait()

    for m in range(DEPTH - LAG):
        gather_start(m, m)

    def group(g, _):
        for b in range(DEPTH):
            j = g * DEPTH + b
            sl = (b - LAG) % DEPTH
            if b >= LAG:
                store_wait(sl)
                @pl.when(g < GROUPS - 1)
                def _():
                    gather_start(j + DEPTH - LAG, sl)
            else:
                @pl.when(g >= 1)
                def _():
                    store_wait(sl)
                gather_start(j + DEPTH - LAG, sl)
            gather_wait(b)
            store_start(j, b)
        return 0

    lax.fori_loop(0, GROUPS, group, 0)

    for i in range(LAG):
        store_wait((CH_PER_W - LAG + i) % DEPTH)


@jax.jit
def _gather(idx2d, table):
    mesh = plsc.VectorSubcoreMesh(core_axis_name="c", subcore_axis_name="s")
    f = pl.kernel(
        _body,
        out_type=jax.ShapeDtypeStruct((NUM_TOKENS, EMB_D), jnp.float32),
        mesh=mesh,
        scratch_types=(
            [pltpu.VMEM((CH_PER_W, CHUNK), jnp.int32)]
            + [pltpu.VMEM((CHUNK, EMB_D), jnp.float32) for _ in range(DEPTH)]
            + [pltpu.SemaphoreType.DMA for _ in range(2 * DEPTH)]
        ),
        compiler_params=pltpu.CompilerParams(use_tc_tiling_on_sc=False),
    )
    return f(idx2d, table)


def kernel(token_ids, embedding_matrix):
    idx2d = token_ids.reshape(NUM_CHUNKS, CHUNK).astype(jnp.int32)
    out = _gather(idx2d, embedding_matrix)
    return out.reshape(token_ids.shape + (EMB_D,))


# speedup vs baseline: 1.1133x; 1.0004x over previous
import jax
import jax.numpy as jnp
from jax import lax
from jax.experimental import pallas as pl
from jax.experimental.pallas import tpu as pltpu
from jax.experimental.pallas import tpu_sc as plsc

NUM_TOKENS = 16384 * 50
CHUNK = 128
NUM_CHUNKS = NUM_TOKENS // CHUNK
NC, NS = 2, 16
NW = NC * NS
CH_PER_W = NUM_CHUNKS // NW
EMB_D = 32
DEPTH = 8
LAG = 4
GROUPS = CH_PER_W // DEPTH


def _body(idx_hbm, table_hbm, out_hbm, idx_v, *rest):
    rows = rest[:DEPTH]
    gsem = rest[DEPTH:2 * DEPTH]
    ssem = rest[2 * DEPTH:3 * DEPTH]
    wid = lax.axis_index("s") * NC + lax.axis_index("c")
    ch_base = wid * CH_PER_W
    tok_base = ch_base * CHUNK
    pltpu.sync_copy(idx_hbm.at[pl.ds(tok_base, CH_PER_W * CHUNK)], idx_v)

    def gather_start(chunk, slot):
        pltpu.async_copy(
            table_hbm.at[idx_v.at[pl.ds(chunk * CHUNK, CHUNK)]], rows[slot],
            gsem[slot])

    def gather_wait(slot):
        pltpu.make_async_copy(table_hbm.at[pl.ds(0, CHUNK)], rows[slot],
                              gsem[slot]).wait()

    def store_start(chunk, slot):
        pltpu.async_copy(rows[slot],
                         out_hbm.at[pl.ds(tok_base + chunk * CHUNK, CHUNK)],
                         ssem[slot])

    def store_wait(slot):
        pltpu.make_async_copy(rows[slot], out_hbm.at[pl.ds(0, CHUNK)],
                              ssem[slot]).wait()

    for m in range(DEPTH - LAG):
        gather_start(m, m)

    def group(g, _):
        for b in range(DEPTH):
            j = g * DEPTH + b
            sl = (b - LAG) % DEPTH
            if b >= LAG:
                store_wait(sl)
                @pl.when(g < GROUPS - 1)
                def _():
                    gather_start(j + DEPTH - LAG, sl)
            else:
                @pl.when(g >= 1)
                def _():
                    store_wait(sl)
                gather_start(j + DEPTH - LAG, sl)
            gather_wait(b)
            store_start(j, b)
        return 0

    lax.fori_loop(0, GROUPS, group, 0)

    for i in range(LAG):
        store_wait((CH_PER_W - LAG + i) % DEPTH)


@jax.jit
def _gather(idx2d, table):
    mesh = plsc.VectorSubcoreMesh(core_axis_name="c", subcore_axis_name="s")
    f = pl.kernel(
        _body,
        out_type=jax.ShapeDtypeStruct((NUM_TOKENS, EMB_D), jnp.float32),
        mesh=mesh,
        scratch_types=(
            [pltpu.VMEM((CH_PER_W * CHUNK,), jnp.int32)]
            + [pltpu.VMEM((CHUNK, EMB_D), jnp.float32) for _ in range(DEPTH)]
            + [pltpu.SemaphoreType.DMA for _ in range(2 * DEPTH)]
        ),
        compiler_params=pltpu.CompilerParams(use_tc_tiling_on_sc=False),
    )
    return f(idx2d, table)


def kernel(token_ids, embedding_matrix):
    idx1d = token_ids.reshape(NUM_TOKENS).astype(jnp.int32)
    out = _gather(idx1d, embedding_matrix)
    return out.reshape(token_ids.shape + (EMB_D,))
